# baseline (speedup 1.0000x reference)
import functools

import jax
import jax.numpy as jnp
from jax import lax
from jax.experimental import pallas as pl
from jax.experimental.pallas import tpu as pltpu
from jax.experimental.pallas import tpu_sc as plsc

_S_TC = 1792
_CHUNK = 32
_L = 16


def _tc_body(ids_ref, x_ref, out_ref):
    w = (ids_ref[0] != 0).astype(jnp.float32)
    s = jax.lax.dot_general(
        w, x_ref[0], (((1,), (0,)), ((), ())),
        preferred_element_type=jnp.float32)
    out_ref[...] = s.reshape(out_ref.shape)


def _sc_body(s_total, s_tc, d, x_hbm, w_hbm, out_hbm,
             buf, wtile, acc, sem0, sem1):
    rows_per_tile = (s_total - s_tc) // 2
    nchunks = rows_per_tile // _CHUNK
    wid = lax.axis_index("s") * 2 + lax.axis_index("c")
    b = wid // 2
    half = wid % 2
    row0 = b * s_total + s_tc + half * rows_per_tile
    wrow0 = b * (s_total - s_tc) + half * rows_per_tile

    pltpu.sync_copy(w_hbm.at[pl.ds(wrow0, rows_per_tile)], wtile)
    zeros = jnp.zeros((_L,), jnp.float32)
    for dd in range(d // _L):
        acc[pl.ds(dd * _L, _L)] = zeros

    sems = (sem0, sem1)
    cps = [None, None]
    cps[0] = pltpu.async_copy(
        x_hbm.at[pl.ds(row0, _CHUNK)], buf.at[0], sems[0])
    for c in range(nchunks):
        slot = c % 2
        if c + 1 < nchunks:
            nslot = (c + 1) % 2
            cps[nslot] = pltpu.async_copy(
                x_hbm.at[pl.ds(row0 + (c + 1) * _CHUNK, _CHUNK)],
                buf.at[nslot], sems[nslot])
        cps[slot].wait()

        def row_body(r, carry, c=c, slot=slot):
            wf = wtile[c * _CHUNK + r]
            for dd in range(d // _L):
                v = buf[slot, r, pl.ds(dd * _L, _L)] * wf
                plsc.addupdate(acc.at[pl.ds(dd * _L, _L)], v)
            return carry

        lax.fori_loop(0, _CHUNK, row_body, 0)

    pltpu.sync_copy(acc, out_hbm.at[wid])


def _combine_body(ids_ref, tc_ref, sc_ref, out_ref):
    w = (ids_ref[:, 0, :] != 0).astype(jnp.float32)
    cnt = jnp.sum(w, axis=1, keepdims=True)
    tot = (tc_ref[...].reshape(out_ref.shape)
           + sc_ref[:, 0, :] + sc_ref[:, 1, :])
    out_ref[...] = tot / cnt


def kernel(x, input):
    B, S, D = x.shape
    ids = input.astype(jnp.int32)
    ids3 = ids.reshape(B, 1, S)

    tc_sum = pl.pallas_call(
        _tc_body,
        grid=(B,),
        in_specs=[
            pl.BlockSpec((1, 1, _S_TC), lambda b: (b, 0, 0)),
            pl.BlockSpec((1, _S_TC, D), lambda b: (b, 0, 0)),
        ],
        out_specs=pl.BlockSpec((1, 8, D // 8), lambda b: (b, 0, 0)),
        out_shape=jax.ShapeDtypeStruct((B, 8, D // 8), jnp.float32),
    )(ids3, x)

    mesh = plsc.VectorSubcoreMesh(core_axis_name="c", subcore_axis_name="s",
                                  num_cores=2, num_subcores=16)
    sc_fn = pl.kernel(
        functools.partial(_sc_body, S, _S_TC, D),
        out_type=jax.ShapeDtypeStruct((2 * B, D), jnp.float32),
        mesh=mesh,
        scratch_types=[
            pltpu.VMEM((2, _CHUNK, D), jnp.float32),
            pltpu.VMEM(((S - _S_TC) // 2, _L), jnp.float32),
            pltpu.VMEM((D,), jnp.float32),
            pltpu.SemaphoreType.DMA,
            pltpu.SemaphoreType.DMA,
        ],
    )
    s_sc = S - _S_TC
    w_exp = jnp.broadcast_to(
        (ids[:, _S_TC:] != 0).astype(jnp.float32).reshape(B, s_sc, 1),
        (B, s_sc, _L)).reshape(B * s_sc, _L)
    sc_sum = sc_fn(x.reshape(B * S, D), w_exp)

    out = pl.pallas_call(
        _combine_body,
        in_specs=[
            pl.BlockSpec((B, 1, S), lambda: (0, 0, 0)),
            pl.BlockSpec((B, 8, D // 8), lambda: (0, 0, 0)),
            pl.BlockSpec((B, 2, D), lambda: (0, 0, 0)),
        ],
        out_specs=pl.BlockSpec((B, D), lambda: (0, 0)),
        out_shape=jax.ShapeDtypeStruct((B, D), jnp.float32),
    )(ids3, tc_sum, sc_sum.reshape(B, 2, D))
    return out


# speedup vs baseline: 1.0980x; 1.0980x over previous
import functools

import jax
import jax.numpy as jnp
from jax import lax
from jax.experimental import pallas as pl
from jax.experimental.pallas import tpu as pltpu
from jax.experimental.pallas import tpu_sc as plsc

_S_TC = 1792
_CHUNK = 32
_L = 16
_DB = 8


def _tc_body(ids_ref, x_ref, out_ref):
    w = (ids_ref[0] != 0).astype(jnp.float32)
    s = jax.lax.dot_general(
        w, x_ref[0], (((1,), (0,)), ((), ())),
        preferred_element_type=jnp.float32)
    out_ref[...] = s.reshape(out_ref.shape)


def _sc_body(s_total, s_tc, d, x_hbm, w_hbm, out_hbm,
             buf, wtile, acc, sem0, sem1):
    rows_per_tile = (s_total - s_tc) // 2
    nchunks = rows_per_tile // _CHUNK
    wid = lax.axis_index("s") * 2 + lax.axis_index("c")
    b = wid // 2
    half = wid % 2
    row0 = b * s_total + s_tc + half * rows_per_tile
    wrow0 = b * (s_total - s_tc) + half * rows_per_tile

    pltpu.sync_copy(w_hbm.at[pl.ds(wrow0, rows_per_tile)], wtile)

    sems = (sem0, sem1)
    cps = [None, None]
    cps[0] = pltpu.async_copy(
        x_hbm.at[pl.ds(row0, _CHUNK)], buf.at[0], sems[0])
    zero = jnp.zeros((_L,), jnp.float32)
    for c in range(nchunks):
        slot = c % 2
        if c + 1 < nchunks:
            nslot = (c + 1) % 2
            cps[nslot] = pltpu.async_copy(
                x_hbm.at[pl.ds(row0 + (c + 1) * _CHUNK, _CHUNK)],
                buf.at[nslot], sems[nslot])
        cps[slot].wait()

        for db in range(d // _L // _DB):
            def row_body(r, accs, c=c, slot=slot, db=db):
                wf = wtile[c * _CHUNK + r]
                return tuple(
                    a + buf[slot, r, pl.ds((db * _DB + k) * _L, _L)] * wf
                    for k, a in enumerate(accs))

            accs = plsc.parallel_loop(
                0, _CHUNK, carry=(zero,) * _DB, unroll=2)(row_body)
            for k in range(_DB):
                sl = pl.ds((db * _DB + k) * _L, _L)
                if c == 0:
                    acc[sl] = accs[k]
                else:
                    plsc.addupdate(acc.at[sl], accs[k])

    pltpu.sync_copy(acc, out_hbm.at[wid])


def _combine_body(ids_ref, tc_ref, sc_ref, out_ref):
    w = (ids_ref[:, 0, :] != 0).astype(jnp.float32)
    cnt = jnp.sum(w, axis=1, keepdims=True)
    tot = (tc_ref[...].reshape(out_ref.shape)
           + sc_ref[:, 0, :] + sc_ref[:, 1, :])
    out_ref[...] = tot / cnt


def kernel(x, input):
    B, S, D = x.shape
    ids = input.astype(jnp.int32)
    ids3 = ids.reshape(B, 1, S)

    tc_sum = pl.pallas_call(
        _tc_body,
        grid=(B,),
        in_specs=[
            pl.BlockSpec((1, 1, _S_TC), lambda b: (b, 0, 0)),
            pl.BlockSpec((1, _S_TC, D), lambda b: (b, 0, 0)),
        ],
        out_specs=pl.BlockSpec((1, 8, D // 8), lambda b: (b, 0, 0)),
        out_shape=jax.ShapeDtypeStruct((B, 8, D // 8), jnp.float32),
    )(ids3, x)

    mesh = plsc.VectorSubcoreMesh(core_axis_name="c", subcore_axis_name="s",
                                  num_cores=2, num_subcores=16)
    sc_fn = pl.kernel(
        functools.partial(_sc_body, S, _S_TC, D),
        out_type=jax.ShapeDtypeStruct((2 * B, D), jnp.float32),
        mesh=mesh,
        scratch_types=[
            pltpu.VMEM((2, _CHUNK, D), jnp.float32),
            pltpu.VMEM(((S - _S_TC) // 2, _L), jnp.float32),
            pltpu.VMEM((D,), jnp.float32),
            pltpu.SemaphoreType.DMA,
            pltpu.SemaphoreType.DMA,
        ],
    )
    s_sc = S - _S_TC
    w_exp = jnp.broadcast_to(
        (ids[:, _S_TC:] != 0).astype(jnp.float32).reshape(B, s_sc, 1),
        (B, s_sc, _L)).reshape(B * s_sc, _L)
    sc_sum = sc_fn(x.reshape(B * S, D), w_exp)

    out = pl.pallas_call(
        _combine_body,
        in_specs=[
            pl.BlockSpec((B, 1, S), lambda: (0, 0, 0)),
            pl.BlockSpec((B, 8, D // 8), lambda: (0, 0, 0)),
            pl.BlockSpec((B, 2, D), lambda: (0, 0, 0)),
        ],
        out_specs=pl.BlockSpec((B, D), lambda: (0, 0)),
        out_shape=jax.ShapeDtypeStruct((B, D), jnp.float32),
    )(ids3, tc_sum, sc_sum.reshape(B, 2, D))
    return out
